# baseline (speedup 1.0000x reference)
import functools

import jax
import jax.numpy as jnp
from jax import lax
from jax.experimental import pallas as pl
from jax.experimental.pallas import tpu as pltpu
from jax.experimental.pallas import tpu_sc as plsc

_NUM_EMBEDDINGS = 8192
_EMBEDDING_DIM = 32
_COMMITMENT_COST = 0.25
_TILE = 128


def _dist_argmin_kernel(x_ref, w_ref, xsq_ref, wsq_ref, idx_ref, dsum_ref):
    x = x_ref[...]
    w = w_ref[...]
    c = lax.dot_general(x, w, (((1,), (1,)), ((), ())),
                        preferred_element_type=jnp.float32)
    dist = (xsq_ref[...] + wsq_ref[...]) - 2.0 * c
    m = jnp.min(dist, axis=1, keepdims=True)
    ii = lax.broadcasted_iota(jnp.int32, dist.shape, 1)
    idx = jnp.min(jnp.where(dist == m, ii, jnp.int32(2**30)),
                  axis=1, keepdims=True)
    idx_ref[...] = idx

    @pl.when(pl.program_id(0) == 0)
    def _():
        dsum_ref[...] = jnp.zeros_like(dsum_ref)

    dsum_ref[...] += jnp.sum(m)


def _dist_argmin(flat, weight, xsq, wsq_row):
    ntok = flat.shape[0]
    k = weight.shape[0]
    grid = (ntok // _TILE,)
    return pl.pallas_call(
        _dist_argmin_kernel,
        grid=grid,
        in_specs=[
            pl.BlockSpec((_TILE, _EMBEDDING_DIM), lambda i: (i, 0)),
            pl.BlockSpec((k, _EMBEDDING_DIM), lambda i: (0, 0)),
            pl.BlockSpec((_TILE, 1), lambda i: (i, 0)),
            pl.BlockSpec((1, k), lambda i: (0, 0)),
        ],
        out_specs=[
            pl.BlockSpec((_TILE, 1), lambda i: (i, 0)),
            pl.BlockSpec((1, 1), lambda i: (0, 0)),
        ],
        out_shape=[
            jax.ShapeDtypeStruct((ntok, 1), jnp.int32),
            jax.ShapeDtypeStruct((1, 1), jnp.float32),
        ],
    )(flat, weight, xsq, wsq_row)


def _make_sc_gather(v, d, b):
    info = plsc.get_sparse_core_info()
    nw = info.num_cores * info.num_subcores
    assert b % (8 * nw) == 0
    b_per_w = b // nw
    mesh = plsc.VectorSubcoreMesh(core_axis_name="c", subcore_axis_name="s")

    @functools.partial(
        pl.kernel, mesh=mesh,
        out_type=jax.ShapeDtypeStruct((b, d), jnp.float32),
        scratch_types=[
            pltpu.VMEM((b_per_w,), jnp.int32),
            pltpu.VMEM((b_per_w, d), jnp.float32),
            pltpu.SemaphoreType.DMA,
        ],
    )
    def k(table_hbm, idx_hbm, out_hbm, idx_v, rows_v, sem):
        wid = lax.axis_index("s") * info.num_cores + lax.axis_index("c")
        base = wid * b_per_w
        pltpu.sync_copy(idx_hbm.at[pl.ds(base, b_per_w)], idx_v)
        pltpu.async_copy(table_hbm.at[idx_v], rows_v, sem).wait()
        pltpu.sync_copy(rows_v, out_hbm.at[pl.ds(base, b_per_w)])

    return k


def kernel(inputs, weight):
    b, c, l, h, w = inputs.shape
    flat = jnp.transpose(inputs, (0, 2, 3, 4, 1)).reshape(-1, _EMBEDDING_DIM)
    xsq = jnp.sum(flat ** 2, axis=1, keepdims=True)
    wsq = jnp.sum(weight ** 2, axis=1)

    idx2d, dsum = _dist_argmin(flat, weight, xsq, wsq.reshape(1, -1))

    wpad = jnp.pad(weight, ((0, 0), (0, 128 - _EMBEDDING_DIM)))
    gather = _make_sc_gather(weight.shape[0], 128, flat.shape[0])
    quant_flat = gather(wpad, idx2d.reshape(-1))[:, :_EMBEDDING_DIM]

    quantized = jnp.transpose(quant_flat.reshape(b, l, h, w, c),
                              (0, 4, 1, 2, 3))
    mean_sq = dsum[0, 0] / jnp.float32(inputs.size)
    loss = mean_sq + _COMMITMENT_COST * mean_sq
    quantized_st = inputs + lax.stop_gradient(quantized - inputs)
    return (quantized_st, loss, idx2d)


# speedup vs baseline: 1.0801x; 1.0801x over previous
import functools

import jax
import jax.numpy as jnp
from jax import lax
from jax.experimental import pallas as pl
from jax.experimental.pallas import tpu as pltpu
from jax.experimental.pallas import tpu_sc as plsc

_NUM_EMBEDDINGS = 8192
_EMBEDDING_DIM = 32
_COMMITMENT_COST = 0.25
_TILE = 256


def _dist_argmin_kernel(x2_ref, w_ref, xsq_ref, wsq_ref, iota_ref,
                        idx_ref, dsum_ref):
    x2 = x2_ref[...]
    w = w_ref[...]
    c2 = lax.dot_general(x2, w, (((1,), (1,)), ((), ())),
                         preferred_element_type=jnp.float32)
    dist = (xsq_ref[...] + wsq_ref[...]) - c2
    m = jnp.min(dist, axis=1, keepdims=True)
    idxf = jnp.min(jnp.where(dist == m, iota_ref[...], jnp.float32(3.0e38)),
                   axis=1, keepdims=True)
    idx_ref[...] = idxf.astype(jnp.int32)

    @pl.when(pl.program_id(0) == 0)
    def _():
        dsum_ref[...] = jnp.zeros_like(dsum_ref)

    dsum_ref[...] += jnp.sum(m)


def _dist_argmin(flat, weight, xsq, wsq_row):
    ntok = flat.shape[0]
    k = weight.shape[0]
    grid = (ntok // _TILE,)
    iota_row = jnp.arange(k, dtype=jnp.float32).reshape(1, k)
    return pl.pallas_call(
        _dist_argmin_kernel,
        grid=grid,
        in_specs=[
            pl.BlockSpec((_TILE, _EMBEDDING_DIM), lambda i: (i, 0)),
            pl.BlockSpec((k, _EMBEDDING_DIM), lambda i: (0, 0)),
            pl.BlockSpec((_TILE, 1), lambda i: (i, 0)),
            pl.BlockSpec((1, k), lambda i: (0, 0)),
            pl.BlockSpec((1, k), lambda i: (0, 0)),
        ],
        out_specs=[
            pl.BlockSpec((_TILE, 1), lambda i: (i, 0)),
            pl.BlockSpec((1, 1), lambda i: (0, 0)),
        ],
        out_shape=[
            jax.ShapeDtypeStruct((ntok, 1), jnp.int32),
            jax.ShapeDtypeStruct((1, 1), jnp.float32),
        ],
    )(flat, weight, xsq, wsq_row, iota_row)


def _make_sc_gather(v, d, b):
    info = plsc.get_sparse_core_info()
    nw = info.num_cores * info.num_subcores
    assert b % (8 * nw) == 0
    b_per_w = b // nw
    mesh = plsc.VectorSubcoreMesh(core_axis_name="c", subcore_axis_name="s")

    @functools.partial(
        pl.kernel, mesh=mesh,
        out_type=jax.ShapeDtypeStruct((b, d), jnp.float32),
        scratch_types=[
            pltpu.VMEM((b_per_w,), jnp.int32),
            pltpu.VMEM((b_per_w, d), jnp.float32),
            pltpu.SemaphoreType.DMA,
        ],
    )
    def k(table_hbm, idx_hbm, out_hbm, idx_v, rows_v, sem):
        wid = lax.axis_index("s") * info.num_cores + lax.axis_index("c")
        base = wid * b_per_w
        pltpu.sync_copy(idx_hbm.at[pl.ds(base, b_per_w)], idx_v)
        pltpu.async_copy(table_hbm.at[idx_v], rows_v, sem).wait()
        pltpu.sync_copy(rows_v, out_hbm.at[pl.ds(base, b_per_w)])

    return k


def kernel(inputs, weight):
    b, c, l, h, w = inputs.shape
    flat = jnp.transpose(inputs, (0, 2, 3, 4, 1)).reshape(-1, _EMBEDDING_DIM)
    xsq = jnp.sum(flat ** 2, axis=1, keepdims=True)
    wsq = jnp.sum(weight ** 2, axis=1)

    idx2d, dsum = _dist_argmin(flat + flat, weight, xsq, wsq.reshape(1, -1))

    wpad = jnp.pad(weight, ((0, 0), (0, 128 - _EMBEDDING_DIM)))
    gather = _make_sc_gather(weight.shape[0], 128, flat.shape[0])
    quant_flat = gather(wpad, idx2d.reshape(-1))[:, :_EMBEDDING_DIM]

    quantized = jnp.transpose(quant_flat.reshape(b, l, h, w, c),
                              (0, 4, 1, 2, 3))
    mean_sq = dsum[0, 0] / jnp.float32(inputs.size)
    loss = mean_sq + _COMMITMENT_COST * mean_sq
    quantized_st = inputs + lax.stop_gradient(quantized - inputs)
    return (quantized_st, loss, idx2d)


# speedup vs baseline: 1.1387x; 1.0542x over previous
import functools

import jax
import jax.numpy as jnp
from jax import lax
from jax.experimental import pallas as pl
from jax.experimental.pallas import tpu as pltpu
from jax.experimental.pallas import tpu_sc as plsc

_NUM_EMBEDDINGS = 8192
_EMBEDDING_DIM = 32
_COMMITMENT_COST = 0.25
_TILE = 512


def _dist_argmin_kernel(x2_ref, w_ref, xsq_ref, wsq_ref, iota_ref,
                        idx_ref, dsum_ref):
    x2 = x2_ref[...]
    w = w_ref[...]
    c2 = lax.dot_general(x2, w, (((1,), (1,)), ((), ())),
                         preferred_element_type=jnp.float32)
    dist = (xsq_ref[...] + wsq_ref[...]) - c2
    m = jnp.min(dist, axis=1, keepdims=True)
    idxf = jnp.min(jnp.where(dist == m, iota_ref[...], jnp.float32(3.0e38)),
                   axis=1, keepdims=True)
    idx_ref[...] = idxf.astype(jnp.int32)

    @pl.when(pl.program_id(0) == 0)
    def _():
        dsum_ref[...] = jnp.zeros_like(dsum_ref)

    dsum_ref[...] += jnp.sum(m)


def _dist_argmin(flat, weight, xsq, wsq_row):
    ntok = flat.shape[0]
    k = weight.shape[0]
    grid = (ntok // _TILE,)
    iota_row = jnp.arange(k, dtype=jnp.float32).reshape(1, k)
    return pl.pallas_call(
        _dist_argmin_kernel,
        grid=grid,
        in_specs=[
            pl.BlockSpec((_TILE, _EMBEDDING_DIM), lambda i: (i, 0)),
            pl.BlockSpec((k, _EMBEDDING_DIM), lambda i: (0, 0)),
            pl.BlockSpec((_TILE, 1), lambda i: (i, 0)),
            pl.BlockSpec((1, k), lambda i: (0, 0)),
            pl.BlockSpec((1, k), lambda i: (0, 0)),
        ],
        out_specs=[
            pl.BlockSpec((_TILE, 1), lambda i: (i, 0)),
            pl.BlockSpec((1, 1), lambda i: (0, 0)),
        ],
        out_shape=[
            jax.ShapeDtypeStruct((ntok, 1), jnp.int32),
            jax.ShapeDtypeStruct((1, 1), jnp.float32),
        ],
    )(flat, weight, xsq, wsq_row, iota_row)


def _make_sc_gather(v, d, b):
    info = plsc.get_sparse_core_info()
    nw = info.num_cores * info.num_subcores
    assert b % (8 * nw) == 0
    b_per_w = b // nw
    mesh = plsc.VectorSubcoreMesh(core_axis_name="c", subcore_axis_name="s")

    @functools.partial(
        pl.kernel, mesh=mesh,
        out_type=jax.ShapeDtypeStruct((b, d), jnp.float32),
        scratch_types=[
            pltpu.VMEM((b_per_w,), jnp.int32),
            pltpu.VMEM((b_per_w, d), jnp.float32),
            pltpu.SemaphoreType.DMA,
        ],
    )
    def k(table_hbm, idx_hbm, out_hbm, idx_v, rows_v, sem):
        wid = lax.axis_index("s") * info.num_cores + lax.axis_index("c")
        base = wid * b_per_w
        pltpu.sync_copy(idx_hbm.at[pl.ds(base, b_per_w)], idx_v)
        pltpu.async_copy(table_hbm.at[idx_v], rows_v, sem).wait()
        pltpu.sync_copy(rows_v, out_hbm.at[pl.ds(base, b_per_w)])

    return k


def kernel(inputs, weight):
    b, c, l, h, w = inputs.shape
    flat = jnp.transpose(inputs, (0, 2, 3, 4, 1)).reshape(-1, _EMBEDDING_DIM)
    xsq = jnp.sum(flat ** 2, axis=1, keepdims=True)
    wsq = jnp.sum(weight ** 2, axis=1)

    idx2d, dsum = _dist_argmin(flat + flat, weight, xsq, wsq.reshape(1, -1))

    wpad = jnp.pad(weight, ((0, 0), (0, 128 - _EMBEDDING_DIM)))
    gather = _make_sc_gather(weight.shape[0], 128, flat.shape[0])
    quant_flat = gather(wpad, idx2d.reshape(-1))[:, :_EMBEDDING_DIM]

    quantized = jnp.transpose(quant_flat.reshape(b, l, h, w, c),
                              (0, 4, 1, 2, 3))
    mean_sq = dsum[0, 0] / jnp.float32(inputs.size)
    loss = mean_sq + _COMMITMENT_COST * mean_sq
    quantized_st = inputs + lax.stop_gradient(quantized - inputs)
    return (quantized_st, loss, idx2d)


# speedup vs baseline: 1.1635x; 1.0219x over previous
import functools

import jax
import jax.numpy as jnp
from jax import lax
from jax.experimental import pallas as pl
from jax.experimental.pallas import tpu as pltpu
from jax.experimental.pallas import tpu_sc as plsc

_NUM_EMBEDDINGS = 8192
_EMBEDDING_DIM = 32
_COMMITMENT_COST = 0.25
_TILE = 1024


def _dist_argmin_kernel(x2_ref, w_ref, xsq_ref, wsq_ref, iota_ref,
                        idx_ref, dsum_ref):
    x2 = x2_ref[...]
    w = w_ref[...]
    c2 = lax.dot_general(x2, w, (((1,), (1,)), ((), ())),
                         preferred_element_type=jnp.float32)
    dist = (xsq_ref[...] + wsq_ref[...]) - c2
    m = jnp.min(dist, axis=1, keepdims=True)
    idxf = jnp.min(jnp.where(dist == m, iota_ref[...], jnp.float32(3.0e38)),
                   axis=1, keepdims=True)
    idx_ref[...] = idxf.astype(jnp.int32)

    @pl.when(pl.program_id(0) == 0)
    def _():
        dsum_ref[...] = jnp.zeros_like(dsum_ref)

    dsum_ref[...] += jnp.sum(m)


def _dist_argmin(flat, weight, xsq, wsq_row):
    ntok = flat.shape[0]
    k = weight.shape[0]
    grid = (ntok // _TILE,)
    iota_row = jnp.arange(k, dtype=jnp.float32).reshape(1, k)
    return pl.pallas_call(
        _dist_argmin_kernel,
        grid=grid,
        in_specs=[
            pl.BlockSpec((_TILE, _EMBEDDING_DIM), lambda i: (i, 0)),
            pl.BlockSpec((k, _EMBEDDING_DIM), lambda i: (0, 0)),
            pl.BlockSpec((_TILE, 1), lambda i: (i, 0)),
            pl.BlockSpec((1, k), lambda i: (0, 0)),
            pl.BlockSpec((1, k), lambda i: (0, 0)),
        ],
        out_specs=[
            pl.BlockSpec((_TILE, 1), lambda i: (i, 0)),
            pl.BlockSpec((1, 1), lambda i: (0, 0)),
        ],
        out_shape=[
            jax.ShapeDtypeStruct((ntok, 1), jnp.int32),
            jax.ShapeDtypeStruct((1, 1), jnp.float32),
        ],
    )(flat, weight, xsq, wsq_row, iota_row)


def _make_sc_gather(v, d, b):
    info = plsc.get_sparse_core_info()
    nw = info.num_cores * info.num_subcores
    assert b % (8 * nw) == 0
    b_per_w = b // nw
    mesh = plsc.VectorSubcoreMesh(core_axis_name="c", subcore_axis_name="s")

    @functools.partial(
        pl.kernel, mesh=mesh,
        out_type=jax.ShapeDtypeStruct((b, d), jnp.float32),
        scratch_types=[
            pltpu.VMEM((b_per_w,), jnp.int32),
            pltpu.VMEM((b_per_w, d), jnp.float32),
            pltpu.SemaphoreType.DMA,
        ],
    )
    def k(table_hbm, idx_hbm, out_hbm, idx_v, rows_v, sem):
        wid = lax.axis_index("s") * info.num_cores + lax.axis_index("c")
        base = wid * b_per_w
        pltpu.sync_copy(idx_hbm.at[pl.ds(base, b_per_w)], idx_v)
        pltpu.async_copy(table_hbm.at[idx_v], rows_v, sem).wait()
        pltpu.sync_copy(rows_v, out_hbm.at[pl.ds(base, b_per_w)])

    return k


def kernel(inputs, weight):
    b, c, l, h, w = inputs.shape
    flat = jnp.transpose(inputs, (0, 2, 3, 4, 1)).reshape(-1, _EMBEDDING_DIM)
    xsq = jnp.sum(flat ** 2, axis=1, keepdims=True)
    wsq = jnp.sum(weight ** 2, axis=1)

    idx2d, dsum = _dist_argmin(flat + flat, weight, xsq, wsq.reshape(1, -1))

    wpad = jnp.pad(weight, ((0, 0), (0, 128 - _EMBEDDING_DIM)))
    gather = _make_sc_gather(weight.shape[0], 128, flat.shape[0])
    quant_flat = gather(wpad, idx2d.reshape(-1))[:, :_EMBEDDING_DIM]

    quantized = jnp.transpose(quant_flat.reshape(b, l, h, w, c),
                              (0, 4, 1, 2, 3))
    mean_sq = dsum[0, 0] / jnp.float32(inputs.size)
    loss = mean_sq + _COMMITMENT_COST * mean_sq
    quantized_st = inputs + lax.stop_gradient(quantized - inputs)
    return (quantized_st, loss, idx2d)
